# baseline (speedup 1.0000x reference)
import functools

import jax
import jax.numpy as jnp
from jax import lax
from jax.experimental import pallas as pl
from jax.experimental.pallas import tpu as pltpu
from jax.experimental.pallas import tpu_sc as plsc

_B = 16384
_D = 32
_P = 8
_L = 16
_NW = 32
_ROWS = _B // _NW


def _sc_kernel(q_hbm, proj_hbm, w_hbm, b_hbm, out_hbm, q_v, o_v, proj_v, w_v, b_v):
    wid = lax.axis_index("s") * 2 + lax.axis_index("c")
    base = wid * _ROWS

    pltpu.sync_copy(q_hbm.at[pl.ds(base, _ROWS)], q_v)
    pltpu.sync_copy(proj_hbm, proj_v)
    pltpu.sync_copy(w_hbm.at[0], w_v.at[pl.ds(0, _P)])
    pltpu.sync_copy(b_hbm.at[0], b_v)

    w_vec = w_v[...]
    w_s = [w_vec[p] for p in range(_P)]
    m_chunks = []
    for k in range(_D):
        row = []
        for c in range(2):
            acc = w_s[0] * proj_v[0, k, pl.ds(c * _L, _L)]
            for p in range(1, _P):
                acc = acc + w_s[p] * proj_v[p, k, pl.ds(c * _L, _L)]
            row.append(acc)
        m_chunks.append(row)

    bias0 = b_v[pl.ds(0, _L)]
    bias1 = b_v[pl.ds(_L, _L)]

    def _tree_sum(vals):
        while len(vals) > 1:
            vals = [a + b for a, b in zip(vals[0::2], vals[1::2])] + (
                [vals[-1]] if len(vals) % 2 else [])
        return vals[0]

    def body(i, carry):
        for r in range(2):
            row = i * 2 + r
            q0 = q_v[row, pl.ds(0, _L)]
            q1 = q_v[row, pl.ds(_L, _L)]
            t0 = [bias0]
            t1 = [bias1]
            for k in range(_D):
                s = lax.broadcast_in_dim(
                    q0[k] if k < _L else q1[k - _L], (_L,), ())
                t0.append(s * m_chunks[k][0])
                t1.append(s * m_chunks[k][1])
            o_v[row, pl.ds(0, _L)] = _tree_sum(t0)
            o_v[row, pl.ds(_L, _L)] = _tree_sum(t1)
        return carry

    lax.fori_loop(0, _ROWS // 2, body, 0)

    pltpu.sync_copy(o_v, out_hbm.at[pl.ds(base, _ROWS)])


def kernel(query_embedding, projector, W, b):
    mesh = plsc.VectorSubcoreMesh(core_axis_name="c", subcore_axis_name="s")
    k = functools.partial(
        pl.kernel,
        mesh=mesh,
        compiler_params=pltpu.CompilerParams(use_tc_tiling_on_sc=False),
        out_type=jax.ShapeDtypeStruct((_B, _D), jnp.float32),
        scratch_types=[
            pltpu.VMEM((_ROWS, _D), jnp.float32),
            pltpu.VMEM((_ROWS, _D), jnp.float32),
            pltpu.VMEM((_P, _D, _D), jnp.float32),
            pltpu.VMEM((_L,), jnp.float32),
            pltpu.VMEM((_D,), jnp.float32),
        ],
    )(_sc_kernel)
    out = k(query_embedding, projector, W, b)
    return out[:, None, :]


# speedup vs baseline: 5.6331x; 5.6331x over previous
import jax
import jax.numpy as jnp
from jax.experimental import pallas as pl


def _fused_kernel(q_ref, proj_ref, w_ref, b_ref, out_ref):
    wv = w_ref[0, :]
    m = jnp.sum(proj_ref[:] * wv[:, None, None], axis=0)
    q = q_ref[:]
    acc = jax.lax.dot_general(
        q, m, (((1,), (0,)), ((), ())),
        preferred_element_type=jnp.float32,
    )
    out_ref[:] = acc + b_ref[0, :][None, :]


def kernel(query_embedding, projector, W, b):
    B, D = query_embedding.shape
    P = projector.shape[0]
    BLK = 8192
    grid = (B // BLK,)
    out = pl.pallas_call(
        _fused_kernel,
        grid=grid,
        in_specs=[
            pl.BlockSpec((BLK, D), lambda i: (i, 0)),
            pl.BlockSpec((P, D, D), lambda i: (0, 0, 0)),
            pl.BlockSpec((1, P), lambda i: (0, 0)),
            pl.BlockSpec((1, D), lambda i: (0, 0)),
        ],
        out_specs=pl.BlockSpec((BLK, D), lambda i: (i, 0)),
        out_shape=jax.ShapeDtypeStruct((B, D), jnp.float32),
    )(query_embedding, projector, W, b)
    return out[:, None, :]
